# baseline (speedup 1.0000x reference)
import jax
import jax.numpy as jnp
from jax import lax
from jax.experimental import pallas as pl
from jax.experimental.pallas import tpu as pltpu, tpu_sc as plsc

B, J, D = 1024, 26, 1000
ROWS = J * D
NBANDS = ROWS // 8
BWORDS = 8 * B
NW = 32
NB_BIG = 102
W_BIG = NBANDS - 101 * NW
CH_B = 7
CHW = CH_B * BWORDS
NBUF = 2
NCH = 15

_info = plsc.get_sparse_core_info()
NC, NS, L = _info.num_cores, _info.num_subcores, _info.num_lanes


def _sc_body(cf_hbm, out_hbm, cf_v, buf0, buf1, sems):
    wid = lax.axis_index("s") * NC + lax.axis_index("c")
    nb = jnp.where(wid < W_BIG, NB_BIG, NB_BIG - 1)
    blo = NB_BIG * wid - jnp.maximum(wid - W_BIG, 0)
    bhi = blo + nb

    jlo = (blo * 8) // D
    j2 = jnp.minimum(jlo + 1, J - 1)
    pltpu.sync_copy(cf_hbm.at[pl.ds(jlo, 1)], cf_v.at[pl.ds(0, 1)])
    pltpu.sync_copy(cf_hbm.at[pl.ds(j2, 1)], cf_v.at[pl.ds(1, 1)])

    def _zero(buf):
        def _body(i, _):
            z = jnp.zeros((L,), jnp.float32)
            for u in range(8):
                buf[pl.ds((i * 8 + u) * L, L)] = z
            return 0

        lax.fori_loop(0, CHW // (8 * L), _body, 0)

    lane = lax.broadcasted_iota(jnp.int32, (L,), 0)
    ones = jnp.ones((L,), jnp.float32)
    zeros = jnp.zeros((L,), jnp.float32)

    def _chunk_lo(m):
        return jnp.minimum(blo + m * CH_B, bhi - CH_B)

    def _scan(buf, m, val):
        rlo = _chunk_lo(m) * 8
        rhi = rlo + CH_B * 8
        ja = rlo // D - jlo
        jb = (rhi - 1) // D - jlo

        def _pass(jrow):
            def _body(k, _):
                c = k * L + lane
                v = cf_v[jrow, pl.ds(k * L, L)]
                msk = (v >= rlo) & (v < rhi)
                dr = v - rlo
                phys = ((dr >> 3) << 13) + ((dr & 7) << 7) + ((c >> 7) << 10) + (c & 127)
                plsc.store_scatter(buf, [phys], val, mask=msk)
                return 0

            lax.fori_loop(0, B // L, _body, 0)

        _pass(ja)

        @pl.when(jb != ja)
        def _():
            _pass(jb)

    bufs = (buf0, buf1)
    for b in range(NBUF):
        _zero(bufs[b])
        _scan(bufs[b], b, ones)
        dst = out_hbm.at[pl.ds(_chunk_lo(b) * BWORDS, CHW)]
        pltpu.make_async_copy(bufs[b], dst, sems.at[b]).start()
    for mg in range(NBUF, NCH + NBUF - 1, NBUF):
        for b in range(NBUF):
            m = mg + b
            if m >= NCH:
                continue
            buf = bufs[b]
            dst = out_hbm.at[pl.ds(_chunk_lo(m) * BWORDS, CHW)]
            pltpu.make_async_copy(buf, dst, sems.at[b]).wait()
            _scan(buf, m - NBUF, zeros)
            _scan(buf, m, ones)
            pltpu.make_async_copy(buf, dst, sems.at[b]).start()
    for b in range(NBUF):
        m = max(mm for mm in range(NCH) if mm % NBUF == b)
        dst = out_hbm.at[pl.ds(_chunk_lo(m) * BWORDS, CHW)]
        pltpu.make_async_copy(bufs[b], dst, sems.at[b]).wait()


def kernel(x):
    cf = x.T + jnp.arange(J, dtype=x.dtype)[:, None] * D
    mesh = plsc.VectorSubcoreMesh(core_axis_name="c", subcore_axis_name="s")
    f = pl.kernel(
        _sc_body,
        out_type=jax.ShapeDtypeStruct((ROWS * B,), jnp.float32),
        mesh=mesh,
        scratch_types=[
            pltpu.VMEM((2, B), jnp.int32),
            pltpu.VMEM((CHW,), jnp.float32),
            pltpu.VMEM((CHW,), jnp.float32),
            pltpu.SemaphoreType.DMA((NBUF,)),
        ],
        compiler_params=pltpu.CompilerParams(needs_layout_passes=False),
    )
    o = f(cf)
    return o.reshape(NBANDS, 8, 8, 128).transpose(0, 2, 1, 3).reshape(ROWS, B).T


# speedup vs baseline: 1.0256x; 1.0256x over previous
import jax
import jax.numpy as jnp
from jax import lax
from jax.experimental import pallas as pl
from jax.experimental.pallas import tpu as pltpu, tpu_sc as plsc

B, J, D = 1024, 26, 1000
ROWS = J * D
NBANDS = ROWS // 8
BWORDS = 8 * B
NW = 32
NB_BIG = 102
W_BIG = NBANDS - 101 * NW
CH_B = 7
CHW = CH_B * BWORDS
NBUF = 2
NCH = 15

_info = plsc.get_sparse_core_info()
NC, NS, L = _info.num_cores, _info.num_subcores, _info.num_lanes


def _sc_body(cf_hbm, out_hbm, cf_v, buf0, buf1, sems):
    wid = lax.axis_index("s") * NC + lax.axis_index("c")
    nb = jnp.where(wid < W_BIG, NB_BIG, NB_BIG - 1)
    blo = NB_BIG * wid - jnp.maximum(wid - W_BIG, 0)
    bhi = blo + nb

    jlo = (blo * 8) // D
    j2 = jnp.minimum(jlo + 1, J - 1)
    pltpu.sync_copy(cf_hbm.at[pl.ds(jlo * B, B)], cf_v.at[pl.ds(0, B)])
    pltpu.sync_copy(cf_hbm.at[pl.ds(j2 * B, B)], cf_v.at[pl.ds(B, B)])

    def _zero(buf):
        def _body(i, _):
            z = jnp.zeros((L,), jnp.float32)
            for u in range(8):
                buf[pl.ds((i * 8 + u) * L, L)] = z
            return 0

        lax.fori_loop(0, CHW // (8 * L), _body, 0)

    lane = lax.broadcasted_iota(jnp.int32, (L,), 0)
    ones = jnp.ones((L,), jnp.float32)
    zeros = jnp.zeros((L,), jnp.float32)

    def _chunk_lo(m):
        return jnp.minimum(blo + m * CH_B, bhi - CH_B)

    def _scan(buf, m, val):
        rlo = _chunk_lo(m) * 8
        rhi = rlo + CH_B * 8
        ja = rlo // D - jlo
        jb = (rhi - 1) // D - jlo

        def _pass(jrow):
            base = jrow * B

            def _body(k, _):
                for u in range(2):
                    kk = k * 2 + u
                    c = kk * L + lane
                    v = cf_v[pl.ds(base + kk * L, L)]
                    msk = (v >= rlo) & (v < rhi)
                    dr = v - rlo
                    phys = (
                        ((dr >> 3) << 13) + ((dr & 7) << 7) + ((c >> 7) << 10) + (c & 127)
                    )
                    plsc.store_scatter(buf, [phys], val, mask=msk)
                return 0

            lax.fori_loop(0, B // (2 * L), _body, 0)

        _pass(ja)

        @pl.when(jb != ja)
        def _():
            _pass(jb)

    bufs = (buf0, buf1)
    for b in range(NBUF):
        _zero(bufs[b])
        _scan(bufs[b], b, ones)
        dst = out_hbm.at[pl.ds(_chunk_lo(b) * BWORDS, CHW)]
        pltpu.make_async_copy(bufs[b], dst, sems.at[b]).start()
    for mg in range(NBUF, NCH + NBUF - 1, NBUF):
        for b in range(NBUF):
            m = mg + b
            if m >= NCH:
                continue
            buf = bufs[b]
            dst = out_hbm.at[pl.ds(_chunk_lo(m) * BWORDS, CHW)]
            pltpu.make_async_copy(buf, dst, sems.at[b]).wait()
            _scan(buf, m - NBUF, zeros)
            _scan(buf, m, ones)
            pltpu.make_async_copy(buf, dst, sems.at[b]).start()
    for b in range(NBUF):
        m = max(mm for mm in range(NCH) if mm % NBUF == b)
        dst = out_hbm.at[pl.ds(_chunk_lo(m) * BWORDS, CHW)]
        pltpu.make_async_copy(bufs[b], dst, sems.at[b]).wait()


def kernel(x):
    cf = (x.T + jnp.arange(J, dtype=x.dtype)[:, None] * D).reshape(-1)
    mesh = plsc.VectorSubcoreMesh(core_axis_name="c", subcore_axis_name="s")
    f = pl.kernel(
        _sc_body,
        out_type=jax.ShapeDtypeStruct((ROWS * B,), jnp.float32),
        mesh=mesh,
        scratch_types=[
            pltpu.VMEM((2 * B,), jnp.int32),
            pltpu.VMEM((CHW,), jnp.float32),
            pltpu.VMEM((CHW,), jnp.float32),
            pltpu.SemaphoreType.DMA((NBUF,)),
        ],
        compiler_params=pltpu.CompilerParams(needs_layout_passes=False),
    )
    o = f(cf)
    return o.reshape(NBANDS, 8, 8, 128).transpose(0, 2, 1, 3).reshape(ROWS, B).T
